# baseline (speedup 1.0000x reference)
import dataclasses
import functools

import jax
import jax.numpy as jnp
from jax import lax
from jax.experimental import pallas as pl
from jax.experimental.pallas import tpu as pltpu
from jax.experimental.pallas import tpu_sc as plsc

_B, _N, _D = 16384, 7, 256
_C = 64
_NC, _NS, _L = 2, 16, 16
_NW = _NC * _NS
_BPW = _B // _NW
_CH = 128
_DP = 257
_MR = 72


def _table_kernel(rank_ref, suit_ref, card_ref, w_ref, m_ref):
    row = lax.broadcasted_iota(jnp.int32, (_C, 1), 0)
    valid = row < 52
    ranks = row % 13
    suits = row // 13
    oh_r = jnp.where(
        (ranks == lax.broadcasted_iota(jnp.int32, (_C, 16), 1)) & valid,
        1.0, 0.0)
    oh_s = jnp.where(
        (suits == lax.broadcasted_iota(jnp.int32, (_C, 8), 1)) & valid,
        1.0, 0.0)
    t = (
        lax.dot_general(oh_r, rank_ref[...], (((1,), (0,)), ((), ())),
                        preferred_element_type=jnp.float32)
        + lax.dot_general(oh_s, suit_ref[...], (((1,), (0,)), ((), ())),
                          preferred_element_type=jnp.float32)
        + card_ref[...]
    )
    m_ref[...] = lax.dot_general(
        t, w_ref[...], (((1,), (1,)), ((), ())),
        preferred_element_type=jnp.float32)


def _build_table(rank_emb, suit_emb, card_emb, W):
    rank_pad = jnp.zeros((16, _D), jnp.float32).at[:13].set(rank_emb)
    suit_pad = jnp.zeros((8, _D), jnp.float32).at[:4].set(suit_emb)
    card_pad = jnp.zeros((_C, _D), jnp.float32).at[:52].set(card_emb)
    return pl.pallas_call(
        _table_kernel,
        out_shape=jax.ShapeDtypeStruct((_C, _D), jnp.float32),
    )(rank_pad, suit_pad, card_pad, W)


_NIDX = 128
_RPC = _NIDX // (_N + 1)
_NCHUNK = _BPW // _RPC
_OBR = 128
_CPB = _OBR // _RPC


def _sc_body(cards_hbm, m_hbm, out_hbm, cards_v, g0, g1, ob, sg0, sg1):
    c = lax.axis_index("c")
    s = lax.axis_index("s")
    wid = s * _NC + c
    pltpu.sync_copy(cards_hbm.at[wid], cards_v)
    pltpu.async_copy(m_hbm.at[cards_v.at[0]], g0, sg0)

    @pl.loop(0, _NCHUNK // _CPB)
    def _blk(obi):
        for sub in range(_CPB):
            gbuf = g0 if sub % 2 == 0 else g1
            gsem = sg0 if sub % 2 == 0 else sg1
            nbuf = g1 if sub % 2 == 0 else g0
            nsem = sg1 if sub % 2 == 0 else sg0
            ch = obi * _CPB + sub

            @pl.when(ch + 1 < _NCHUNK)
            def _():
                pltpu.async_copy(m_hbm.at[cards_v.at[ch + 1]], nbuf, nsem)

            pltpu.make_async_copy(m_hbm.at[cards_v.at[ch]], gbuf, gsem).wait()

            @plsc.parallel_loop(0, _RPC)
            def _row(j):
                base = j * (_N + 1)
                ro = sub * _RPC + j
                for k in range(_D // _L):
                    sl = pl.ds(k * _L, _L)
                    a0 = gbuf[base + 0, sl] + gbuf[base + 1, sl]
                    a1 = gbuf[base + 2, sl] + gbuf[base + 3, sl]
                    a2 = gbuf[base + 4, sl] + gbuf[base + 5, sl]
                    a3 = gbuf[base + 6, sl] + gbuf[base + 7, sl]
                    ob[ro, sl] = jnp.maximum((a0 + a1) + (a2 + a3), 0.0)

        pltpu.sync_copy(ob, out_hbm.at[pl.ds(wid * _BPW + obi * _OBR, _OBR)])


def _sc_call(cards_sc, m):
    mesh = plsc.VectorSubcoreMesh(core_axis_name="c", subcore_axis_name="s")
    cp = pltpu.CompilerParams()
    if "needs_layout_passes" in pltpu.CompilerParams.__dataclass_fields__:
        cp = dataclasses.replace(cp, needs_layout_passes=False)
    run = pl.kernel(
        _sc_body,
        mesh=mesh,
        compiler_params=cp,
        out_type=jax.ShapeDtypeStruct((_B, _D), jnp.float32),
        scratch_types=[
            pltpu.VMEM((_NCHUNK, _NIDX), jnp.int32),
            pltpu.VMEM((_NIDX, _D), jnp.float32),
            pltpu.VMEM((_NIDX, _D), jnp.float32),
            pltpu.VMEM((_OBR, _D), jnp.float32),
            pltpu.SemaphoreType.DMA,
            pltpu.SemaphoreType.DMA,
        ],
    )
    return run(cards_sc, m)


def kernel(cards, rank_emb, suit_emb, card_emb, W, b):
    m = _build_table(rank_emb, suit_emb, card_emb, W)
    m_pad = jnp.zeros((_MR, _D), jnp.float32).at[:_C].set(m).at[_C].set(b)
    cards_aug = jnp.concatenate(
        [cards.reshape(_NW, _BPW, _N),
         jnp.full((_NW, _BPW, 1), _C, jnp.int32)], axis=2)
    cards_sc = cards_aug.reshape(_NW, _NCHUNK, _NIDX)
    return _sc_call(cards_sc, m_pad)


# speedup vs baseline: 25.7575x; 25.7575x over previous
import functools

import jax
import jax.numpy as jnp
from jax.experimental import pallas as pl
from jax.experimental.pallas import tpu as pltpu

_B, _N, _D = 16384, 7, 256
_C = 64


def _table_kernel(rank_ref, suit_ref, card_ref, w_ref, m_ref):
    row = jax.lax.broadcasted_iota(jnp.int32, (_C, 1), 0)
    valid = row < 52
    ranks = row % 13
    suits = row // 13
    oh_r = jnp.where(
        (ranks == jax.lax.broadcasted_iota(jnp.int32, (_C, 16), 1)) & valid,
        1.0, 0.0)
    oh_s = jnp.where(
        (suits == jax.lax.broadcasted_iota(jnp.int32, (_C, 8), 1)) & valid,
        1.0, 0.0)
    t = (
        jax.lax.dot_general(oh_r, rank_ref[...],
                            (((1,), (0,)), ((), ())),
                            preferred_element_type=jnp.float32)
        + jax.lax.dot_general(oh_s, suit_ref[...],
                              (((1,), (0,)), ((), ())),
                              preferred_element_type=jnp.float32)
        + card_ref[...]
    )
    m_ref[...] = jax.lax.dot_general(
        t, w_ref[...], (((1,), (1,)), ((), ())),
        preferred_element_type=jnp.float32)


def _build_table(rank_emb, suit_emb, card_emb, W):
    rank_pad = jnp.zeros((16, _D), jnp.float32).at[:13].set(rank_emb)
    suit_pad = jnp.zeros((8, _D), jnp.float32).at[:4].set(suit_emb)
    card_pad = jnp.zeros((_C, _D), jnp.float32).at[:52].set(card_emb)
    return pl.pallas_call(
        _table_kernel,
        out_shape=jax.ShapeDtypeStruct((_C, _D), jnp.float32),
    )(rank_pad, suit_pad, card_pad, W)


def _main_kernel(cards_ref, m_ref, b_ref, out_ref):
    cards = cards_ref[...]
    bins = jax.lax.broadcasted_iota(jnp.int32, (cards.shape[0], _C), 1)
    counts = jnp.zeros((cards.shape[0], _C), jnp.float32)
    for n in range(_N):
        counts += jnp.where(cards[:, n:n + 1] == bins, 1.0, 0.0)
    acc = jax.lax.dot_general(
        counts, m_ref[...], (((1,), (0,)), ((), ())),
        preferred_element_type=jnp.float32)
    out_ref[...] = jnp.maximum(acc + b_ref[...], 0.0)


def kernel(cards, rank_emb, suit_emb, card_emb, W, b):
    m = _build_table(rank_emb, suit_emb, card_emb, W)
    blk = 2048
    grid = (_B // blk,)
    return pl.pallas_call(
        _main_kernel,
        grid=grid,
        in_specs=[
            pl.BlockSpec((blk, _N), lambda i: (i, 0)),
            pl.BlockSpec((_C, _D), lambda i: (0, 0)),
            pl.BlockSpec((1, _D), lambda i: (0, 0)),
        ],
        out_specs=pl.BlockSpec((blk, _D), lambda i: (i, 0)),
        out_shape=jax.ShapeDtypeStruct((_B, _D), jnp.float32),
    )(cards, m, b.reshape(1, _D))


# speedup vs baseline: 31.0821x; 1.2067x over previous
import functools

import jax
import jax.numpy as jnp
from jax import lax
from jax.experimental import pallas as pl
from jax.experimental.pallas import tpu as pltpu

_B, _N, _D = 16384, 7, 256
_C = 64
_BLK = 2048


def _fused_kernel(cards_ref, rank_ref, suit_ref, card_ref, w_ref, b_ref,
                  out_ref, m_scr):
    @pl.when(pl.program_id(0) == 0)
    def _build():
        row = lax.broadcasted_iota(jnp.int32, (_C, 1), 0)
        valid = row < 52
        oh_r = jnp.where(
            (row % 13 == lax.broadcasted_iota(jnp.int32, (_C, 16), 1)) & valid,
            1.0, 0.0)
        oh_s = jnp.where(
            (row // 13 == lax.broadcasted_iota(jnp.int32, (_C, 8), 1)) & valid,
            1.0, 0.0)
        rank_pad = jnp.concatenate(
            [rank_ref[...], jnp.zeros((3, _D), jnp.float32)], axis=0)
        suit_pad = jnp.concatenate(
            [suit_ref[...], jnp.zeros((4, _D), jnp.float32)], axis=0)
        card_pad = jnp.concatenate(
            [card_ref[...], jnp.zeros((12, _D), jnp.float32)], axis=0)
        t = (
            lax.dot_general(oh_r, rank_pad, (((1,), (0,)), ((), ())),
                            preferred_element_type=jnp.float32)
            + lax.dot_general(oh_s, suit_pad, (((1,), (0,)), ((), ())),
                              preferred_element_type=jnp.float32)
            + card_pad
        )
        m_scr[...] = lax.dot_general(
            t, w_ref[...], (((1,), (1,)), ((), ())),
            preferred_element_type=jnp.float32)

    cards = cards_ref[...]
    bins = lax.broadcasted_iota(jnp.int32, (_BLK, _C), 1)
    counts = jnp.zeros((_BLK, _C), jnp.float32)
    for n in range(_N):
        counts += jnp.where(cards[:, n:n + 1] == bins, 1.0, 0.0)
    acc = lax.dot_general(
        counts, m_scr[...], (((1,), (0,)), ((), ())),
        preferred_element_type=jnp.float32)
    out_ref[...] = jnp.maximum(acc + b_ref[...], 0.0)


def kernel(cards, rank_emb, suit_emb, card_emb, W, b):
    grid = (_B // _BLK,)
    return pl.pallas_call(
        _fused_kernel,
        grid=grid,
        in_specs=[
            pl.BlockSpec((_BLK, _N), lambda i: (i, 0)),
            pl.BlockSpec((13, _D), lambda i: (0, 0)),
            pl.BlockSpec((4, _D), lambda i: (0, 0)),
            pl.BlockSpec((52, _D), lambda i: (0, 0)),
            pl.BlockSpec((_D, _D), lambda i: (0, 0)),
            pl.BlockSpec((1, _D), lambda i: (0, 0)),
        ],
        out_specs=pl.BlockSpec((_BLK, _D), lambda i: (i, 0)),
        out_shape=jax.ShapeDtypeStruct((_B, _D), jnp.float32),
        scratch_shapes=[pltpu.VMEM((_C, _D), jnp.float32)],
    )(cards, rank_emb, suit_emb, card_emb, W, b.reshape(1, _D))
